# Initial kernel scaffold; baseline (speedup 1.0000x reference)
import jax
import jax.numpy as jnp
from jax.experimental import pallas as pl


def kernel(center, pos, neg, in_embed, out_embed):
    raise NotImplementedError("write your pallas kernel here")


# speedup vs baseline: 1.5920x; 1.5920x over previous
import functools

import jax
import jax.numpy as jnp
from jax import lax
from jax.experimental import pallas as pl
from jax.experimental.pallas import tpu as pltpu
from jax.experimental.pallas import tpu_sc as plsc

V = 1000000
D = 64
B = 16384
K_NEG = 5

_INFO = plsc.get_sparse_core_info()
NC = _INFO.num_cores
NS = _INFO.num_subcores
L = _INFO.num_lanes
NW = NC * NS
BW = B // NW
C = 128
NCHUNK = BW // C
NG = C // L


def _sc_scores(center, pos, neg_flat, in_embed, out_embed):
    mesh = plsc.VectorSubcoreMesh(core_axis_name="c", subcore_axis_name="s")

    @functools.partial(
        pl.kernel,
        out_type=[
            jax.ShapeDtypeStruct((B,), jnp.float32),
            jax.ShapeDtypeStruct((NW, K_NEG, BW), jnp.float32),
        ],
        mesh=mesh,
        compiler_params=pltpu.CompilerParams(needs_layout_passes=False,
                                              use_tc_tiling_on_sc=False),
        scratch_types=[
            pltpu.VMEM((BW,), jnp.int32),
            pltpu.VMEM((BW,), jnp.int32),
            pltpu.VMEM((BW * K_NEG,), jnp.int32),
            pltpu.VMEM((C, D), jnp.float32),
            pltpu.VMEM((C, D), jnp.float32),
            pltpu.VMEM((C * K_NEG, D), jnp.float32),
            pltpu.VMEM((BW,), jnp.float32),
            pltpu.VMEM((K_NEG, BW), jnp.float32),
            pltpu.SemaphoreType.DMA,
        ],
    )
    def k(in_hbm, out_hbm, cidx_hbm, pidx_hbm, nidx_hbm,
          ps_hbm, ns_hbm,
          cidx, pidx, nidx, vrows, prows, nrows, pscore, nscore, sem):
        wid = lax.axis_index("s") * NC + lax.axis_index("c")
        base = wid * BW
        pltpu.sync_copy(cidx_hbm.at[pl.ds(base, BW)], cidx)
        pltpu.sync_copy(pidx_hbm.at[pl.ds(base, BW)], pidx)
        pltpu.sync_copy(nidx_hbm.at[pl.ds(base * K_NEG, BW * K_NEG)], nidx)

        lanes = lax.iota(jnp.int32, L)

        def chunk_body(c, _):
            pltpu.async_copy(
                in_hbm.at[cidx.at[pl.ds(c * C, C)]], vrows, sem)
            pltpu.async_copy(
                out_hbm.at[pidx.at[pl.ds(c * C, C)]], prows, sem)
            pltpu.async_copy(
                out_hbm.at[nidx.at[pl.ds(c * C * K_NEG, C * K_NEG)]],
                nrows, sem)
            pltpu.make_async_copy(
                in_hbm.at[cidx.at[pl.ds(c * C, C)]], vrows, sem).wait()
            pltpu.make_async_copy(
                out_hbm.at[pidx.at[pl.ds(c * C, C)]], prows, sem).wait()
            pltpu.make_async_copy(
                out_hbm.at[nidx.at[pl.ds(c * C * K_NEG, C * K_NEG)]],
                nrows, sem).wait()

            def group_body(g, _):
                row_v = g * L + lanes
                rows_n = [row_v * K_NEG + kk for kk in range(K_NEG)]

                def d_body(d, accs):
                    col = jnp.full((L,), d, jnp.int32)
                    vv = plsc.load_gather(vrows, [row_v, col])
                    up = plsc.load_gather(prows, [row_v, col])
                    out = [accs[0] + vv * up]
                    for kk in range(K_NEG):
                        un = plsc.load_gather(nrows, [rows_n[kk], col])
                        out.append(accs[1 + kk] + vv * un)
                    return tuple(out)

                zero = jnp.zeros((L,), jnp.float32)
                accs = lax.fori_loop(
                    0, D, d_body, (zero,) * (1 + K_NEG), unroll=2)
                off = c * C + g * L
                pscore[pl.ds(off, L)] = accs[0]
                for kk in range(K_NEG):
                    nscore[kk, pl.ds(off, L)] = accs[1 + kk]
                return ()

            lax.fori_loop(0, NG, group_body, ())
            return ()

        lax.fori_loop(0, NCHUNK, chunk_body, ())
        pltpu.sync_copy(pscore, ps_hbm.at[pl.ds(base, BW)])
        pltpu.sync_copy(nscore, ns_hbm.at[wid])

    return k(in_embed, out_embed, center, pos, neg_flat)


def _tc_loss(pos_score, neg_score):
    ps2 = pos_score.reshape(B // 128, 128)
    ns2 = neg_score.reshape(B * K_NEG // 128, 128)

    def body(ps_ref, ns_ref, out_ref):
        lp = jnp.log(jax.nn.sigmoid(ps_ref[...]) + 1e-9)
        ln = jnp.log(jax.nn.sigmoid(-ns_ref[...]) + 1e-9)
        out_ref[0, 0] = -(jnp.sum(lp) / B + jnp.sum(ln) / (B * K_NEG))

    out = pl.pallas_call(
        body,
        out_shape=jax.ShapeDtypeStruct((1, 1), jnp.float32),
        out_specs=pl.BlockSpec(memory_space=pltpu.SMEM),
    )(ps2, ns2)
    return out[0, 0]


def kernel(center, pos, neg, in_embed, out_embed):
    center = center.astype(jnp.int32)
    pos = pos.astype(jnp.int32)
    neg_flat = neg.astype(jnp.int32).reshape(B * K_NEG)
    ps, ns = _sc_scores(center, pos, neg_flat, in_embed, out_embed)
    return _tc_loss(ps, ns)
